# baseline (speedup 1.0000x reference)
import functools

import jax
import jax.numpy as jnp
from jax import lax
from jax.experimental import pallas as pl
from jax.experimental.pallas import tpu as pltpu
from jax.experimental.pallas import tpu_sc as plsc

VOCAB_N = 100000
EMBED_D = 128
BATCH_B = 16384

_info = plsc.get_sparse_core_info()
_NC, _NS, _L = _info.num_cores, _info.num_subcores, _info.num_lanes
_NW = _NC * _NS
_TOK_W = BATCH_B // _NW
_CHUNK = 256
_NCHUNK = _TOK_W // _CHUNK
_GROWS = 128
_NG = _CHUNK // _GROWS


def _sgns_body(x_hbm, t_hbm, in_hbm, out_hbm, o_hbm,
               xi_v, ti_v, a_v, b_v, out_v, sem):
    wid = lax.axis_index("s") * _NC + lax.axis_index("c")
    base = wid * _TOK_W
    pltpu.sync_copy(x_hbm.at[pl.ds(base, _TOK_W)], xi_v)
    pltpu.sync_copy(t_hbm.at[pl.ds(base, _TOK_W)], ti_v)
    lane = lax.iota(jnp.int32, _L)

    for chunk in range(_NCHUNK):
        cbase = chunk * _CHUNK
        copies = []
        for j in range(_NG):
            o = cbase + j * _GROWS
            copies.append(pltpu.async_copy(
                in_hbm.at[xi_v.at[pl.ds(o, _GROWS)]],
                a_v.at[pl.ds(j * _GROWS, _GROWS)], sem))
            copies.append(pltpu.async_copy(
                out_hbm.at[ti_v.at[pl.ds(o, _GROWS)]],
                b_v.at[pl.ds(j * _GROWS, _GROWS)], sem))
        for c in copies:
            c.wait()

        def group_body(g, carry):
            rows = jnp.int32(g * _L) + lane

            def d_body(dd, acc):
                cols = lane * 0 + dd
                va = plsc.load_gather(a_v, [rows, cols])
                vb = plsc.load_gather(b_v, [rows, cols])
                return acc + va * vb

            acc = lax.fori_loop(0, EMBED_D, d_body,
                                jnp.zeros((_L,), jnp.float32))
            out_v[pl.ds(cbase + g * _L, _L)] = 1.0 / (1.0 + jnp.exp(-acc))
            return carry

        lax.fori_loop(0, _CHUNK // _L, group_body, jnp.int32(0))

    pltpu.sync_copy(out_v, o_hbm.at[pl.ds(base, _TOK_W)])


_sgns_call = functools.partial(
    pl.kernel,
    out_type=jax.ShapeDtypeStruct((BATCH_B,), jnp.float32),
    mesh=plsc.VectorSubcoreMesh(core_axis_name="c", subcore_axis_name="s"),
    compiler_params=pltpu.CompilerParams(needs_layout_passes=False),
    scratch_types=[
        pltpu.VMEM((_TOK_W,), jnp.int32),
        pltpu.VMEM((_TOK_W,), jnp.int32),
        pltpu.VMEM((_CHUNK, EMBED_D), jnp.float32),
        pltpu.VMEM((_CHUNK, EMBED_D), jnp.float32),
        pltpu.VMEM((_TOK_W,), jnp.float32),
        pltpu.SemaphoreType.DMA,
    ],
)(_sgns_body)


def kernel(x, t, in_embed, out_embed):
    return _sgns_call(x.astype(jnp.int32), t.astype(jnp.int32),
                      in_embed, out_embed)
